# baseline (speedup 1.0000x reference)
import functools
import math

import jax
import jax.numpy as jnp
from jax import lax
from jax.experimental import pallas as pl
from jax.experimental.pallas import tpu as pltpu
from jax.experimental.pallas import tpu_sc as plsc

HID = 64
EMB = 128

_info = plsc.get_sparse_core_info()
_NC, _NS = _info.num_cores, _info.num_subcores
_NW = _NC * _NS

_CHUNK = 128


def _proj_body(tt_ref, w_ref, b_ref, o_ref):
    g = tt_ref[...]
    h = 0.5 * g * (1.0 + lax.erf(g * (1.0 / math.sqrt(2.0))))
    acc = lax.dot_general(
        h, w_ref[...], (((0,), (0,)), ((), ())),
        preferred_element_type=jnp.float32,
    )
    o_ref[...] = acc + b_ref[...]


def _project_table(table_t, w_t, b):
    v = table_t.shape[1]
    c = 32768
    grid = (pl.cdiv(v, c),)
    return pl.pallas_call(
        _proj_body,
        grid=grid,
        in_specs=[
            pl.BlockSpec((HID, c), lambda i: (0, i)),
            pl.BlockSpec((HID, EMB), lambda i: (0, 0)),
            pl.BlockSpec((1, EMB), lambda i: (0, 0)),
        ],
        out_specs=pl.BlockSpec((c, EMB), lambda i: (i, 0)),
        out_shape=jax.ShapeDtypeStruct((v, EMB), jnp.float32),
    )(table_t, w_t, b.reshape(1, EMB))


_NSTREAM = 2
_STEP = _NSTREAM * _CHUNK


def _sc_gather(proj, idx_flat):
    n = idx_flat.shape[0]
    assert n % (_NW * _STEP) == 0
    per_w = n // _NW
    assert (per_w // _STEP) % 3 == 1 and per_w // _STEP >= 10
    mesh = plsc.VectorSubcoreMesh(core_axis_name="c", subcore_axis_name="s")

    @functools.partial(
        pl.kernel,
        mesh=mesh,
        out_type=jax.ShapeDtypeStruct((n, EMB), jnp.float32),
        scratch_types=[
            pltpu.VMEM((per_w,), jnp.int32),
            pltpu.VMEM((_STEP, EMB), jnp.float32),
            pltpu.VMEM((_STEP, EMB), jnp.float32),
            pltpu.VMEM((_STEP, EMB), jnp.float32),
            pltpu.SemaphoreType.DMA,
            pltpu.SemaphoreType.DMA,
            pltpu.SemaphoreType.DMA,
            pltpu.SemaphoreType.DMA,
            pltpu.SemaphoreType.DMA,
            pltpu.SemaphoreType.DMA,
        ],
        compiler_params=pltpu.CompilerParams(use_tc_tiling_on_sc=True),
    )
    def k(proj_hbm, idx_hbm, out_hbm, idx_all, r0, r1, r2, g0, g1, g2, w0, w1, w2):
        wid = lax.axis_index("s") * _NC + lax.axis_index("c")
        base = wid * per_w
        pltpu.sync_copy(idx_hbm.at[pl.ds(base, per_w)], idx_all)
        rows = (r0, r1, r2)
        gsem = (g0, g1, g2)
        wsem = (w0, w1, w2)
        chunks = per_w // _STEP

        def g_copies(chunk_i, b):
            return [
                pltpu.make_async_copy(
                    proj_hbm.at[
                        idx_all.at[pl.ds(chunk_i * _STEP + j * _CHUNK, _CHUNK)]
                    ],
                    rows[b].at[pl.ds(j * _CHUNK, _CHUNK)],
                    gsem[b],
                )
                for j in range(_NSTREAM)
            ]

        def wb_copy(chunk_i, b):
            return pltpu.make_async_copy(
                rows[b], out_hbm.at[pl.ds(base + chunk_i * _STEP, _STEP)], wsem[b]
            )

        def fire(chunk_i, b):
            for g in g_copies(chunk_i, b):
                g.start()

        def proc(chunk_i, b, wait_prev_wb, do_fire):
            for g in g_copies(chunk_i, b):
                g.wait()
            wb_copy(chunk_i, b).start()
            if wait_prev_wb:
                wb_copy(chunk_i - 1, (b - 1) % 3).wait()
            if do_fire:
                fire(chunk_i + 2, (chunk_i + 2) % 3)

        fire(0, 0)
        fire(1, 1)
        proc(0, 0, False, True)
        proc(1, 1, True, True)
        proc(2, 2, True, True)

        def body(t, carry):
            i = 3 + 3 * t

            def proc_d(chunk_i, b):
                for g in g_copies(chunk_i, b):
                    g.wait()
                wb_copy(chunk_i, b).start()
                wb_copy(chunk_i - 1, (b - 1) % 3).wait()
                fire(chunk_i + 2, (b + 2) % 3)

            proc_d(i, 0)
            proc_d(i + 1, 1)
            proc_d(i + 2, 2)
            return carry

        lax.fori_loop(0, (chunks - 7) // 3, body, jnp.int32(0))

        proc(chunks - 4, 0, True, True)
        proc(chunks - 3, 1, True, True)
        proc(chunks - 2, 2, True, False)
        proc(chunks - 1, 0, True, False)
        wb_copy(chunks - 1, 0).wait()

    return k(proj, idx_flat)


def kernel(x, table, W_proj, b_proj):
    bsz, seq = x.shape
    idx_flat = x.reshape(-1).astype(jnp.int32)
    table_t = jnp.swapaxes(table, 0, 1)
    w_t = jnp.swapaxes(W_proj, 0, 1)
    proj = _project_table(table_t, w_t, b_proj)
    out = _sc_gather(proj, idx_flat)
    return out.reshape(bsz, seq, EMB)
